# baseline (device time: 33933 ns/iter reference)
import jax
import jax.numpy as jnp
from jax import lax
from jax.experimental import pallas as pl
from jax.experimental.pallas import tpu as pltpu

N_DEV = 4


def kernel(x, Wq, Wo, K_ext, V_ext):
    B, Sq, D = x.shape
    H, Dh = K_ext.shape[2], K_ext.shape[3]
    bf16 = jnp.bfloat16
    f32 = jnp.float32
    Wq_r = Wq.reshape(D, H, Dh)

    def body(x_ref, wq_ref, wo_ref, k_ref, v_ref, out_ref,
             recv_ref, send_sems, recv_sems):
        my = lax.axis_index("i")
        p0 = my ^ 1
        p1 = 3 - my

        barrier_sem = pltpu.get_barrier_semaphore()
        for nbr in (p0, p1):
            pl.semaphore_signal(
                barrier_sem, inc=1,
                device_id=(nbr,), device_id_type=pl.DeviceIdType.MESH,
            )
        pl.semaphore_wait(barrier_sem, 2)

        for b in range(B):
            xb = x_ref[b].astype(bf16)
            acc = jnp.zeros((Sq, D), f32)
            for h in range(H):
                wq_h = wq_ref[:, h, :].astype(bf16)
                q = lax.dot_general(
                    xb, wq_h, (((1,), (0,)), ((), ())),
                    preferred_element_type=f32).astype(bf16)
                kh = k_ref[b, :, h, :].astype(bf16)
                s = lax.dot_general(
                    q, kh, (((1,), (1,)), ((), ())),
                    preferred_element_type=f32) * 0.125
                m = jnp.max(s, axis=-1, keepdims=True)
                p = jnp.exp(s - m)
                l = jnp.sum(p, axis=-1, keepdims=True)
                vh = v_ref[b, :, h, :].astype(bf16)
                o = lax.dot_general(
                    p.astype(bf16), vh, (((1,), (0,)), ((), ())),
                    preferred_element_type=f32)
                o = (o / l).astype(bf16)
                wo_h = wo_ref[pl.ds(h * Dh, Dh), :].astype(bf16)
                acc = acc + lax.dot_general(
                    o, wo_h, (((1,), (0,)), ((), ())),
                    preferred_element_type=f32)
            out_ref[b] = acc

        for step, partner in enumerate((p0, p1)):
            rdma = pltpu.make_async_remote_copy(
                src_ref=out_ref,
                dst_ref=recv_ref.at[step],
                send_sem=send_sems.at[step],
                recv_sem=recv_sems.at[step],
                device_id=(partner,),
                device_id_type=pl.DeviceIdType.MESH,
            )
            rdma.start()
            rdma.wait()
            out_ref[...] = out_ref[...] + recv_ref[step]

    return pl.pallas_call(
        body,
        out_shape=jax.ShapeDtypeStruct((B, Sq, D), f32),
        in_specs=[pl.BlockSpec(memory_space=pltpu.VMEM)] * 5,
        out_specs=pl.BlockSpec(memory_space=pltpu.VMEM),
        scratch_shapes=[
            pltpu.VMEM((2, B, Sq, D), f32),
            pltpu.SemaphoreType.DMA((2,)),
            pltpu.SemaphoreType.DMA((2,)),
        ],
        compiler_params=pltpu.CompilerParams(collective_id=0),
    )(x, Wq_r, Wo, K_ext, V_ext)


# device time: 20665 ns/iter; 1.6421x vs baseline; 1.6421x over previous
import jax
import jax.numpy as jnp
from jax import lax
from jax.experimental import pallas as pl
from jax.experimental.pallas import tpu as pltpu

N_DEV = 4


def kernel(x, Wq, Wo, K_ext, V_ext):
    B, Sq, D = x.shape
    H, Dh = K_ext.shape[2], K_ext.shape[3]
    bf16 = jnp.bfloat16
    f32 = jnp.float32

    def body(x_ref, wq_ref, wo_ref, k_ref, v_ref, out_ref,
             acc_ref, recv_ref, send_sems, recv_sems):
        my = lax.axis_index("i")
        p0 = my ^ 1
        p1 = 3 - my

        barrier_sem = pltpu.get_barrier_semaphore()
        for nbr in (p0, p1):
            pl.semaphore_signal(
                barrier_sem, inc=1,
                device_id=(nbr,), device_id_type=pl.DeviceIdType.MESH,
            )
        pl.semaphore_wait(barrier_sem, 2)

        wq = wq_ref[...].astype(bf16)
        wo = wo_ref[...].astype(bf16)
        for b in range(B):
            xb = x_ref[b].astype(bf16)
            qb = lax.dot_general(
                xb, wq, (((1,), (0,)), ((), ())),
                preferred_element_type=f32).astype(bf16)
            heads = []
            for h in range(H):
                q = qb[:, h * Dh:(h + 1) * Dh]
                kh = k_ref[b, :, h, :].astype(bf16)
                s = lax.dot_general(
                    q, kh, (((1,), (1,)), ((), ())),
                    preferred_element_type=f32) * 0.125
                m = jnp.max(s, axis=-1, keepdims=True)
                p = jnp.exp(s - m)
                l = jnp.sum(p, axis=-1, keepdims=True)
                vh = v_ref[b, :, h, :].astype(bf16)
                o = lax.dot_general(
                    p.astype(bf16), vh, (((1,), (0,)), ((), ())),
                    preferred_element_type=f32)
                heads.append((o / l).astype(bf16))
            attn_b = jnp.concatenate(heads, axis=1)
            acc_ref[b] = lax.dot_general(
                attn_b, wo, (((1,), (0,)), ((), ())),
                preferred_element_type=f32).astype(bf16)

        for step, partner in enumerate((p0, p1)):
            rdma = pltpu.make_async_remote_copy(
                src_ref=acc_ref,
                dst_ref=recv_ref.at[step],
                send_sem=send_sems.at[step],
                recv_sem=recv_sems.at[step],
                device_id=(partner,),
                device_id_type=pl.DeviceIdType.MESH,
            )
            rdma.start()
            rdma.wait()
            acc_ref[...] = acc_ref[...] + recv_ref[step]
        out_ref[...] = acc_ref[...].astype(f32)

    return pl.pallas_call(
        body,
        out_shape=jax.ShapeDtypeStruct((B, Sq, D), f32),
        in_specs=[pl.BlockSpec(memory_space=pltpu.VMEM)] * 5,
        out_specs=pl.BlockSpec(memory_space=pltpu.VMEM),
        scratch_shapes=[
            pltpu.VMEM((B, Sq, D), bf16),
            pltpu.VMEM((2, B, Sq, D), bf16),
            pltpu.SemaphoreType.DMA((2,)),
            pltpu.SemaphoreType.DMA((2,)),
        ],
        compiler_params=pltpu.CompilerParams(collective_id=0),
    )(x, Wq, Wo, K_ext, V_ext)
